# baseline (speedup 1.0000x reference)
import functools

import jax
import jax.numpy as jnp
from jax import lax
from jax.experimental import pallas as pl
from jax.experimental.pallas import tpu as pltpu
from jax.experimental.pallas import tpu_sc as plsc

RADIUS2 = 0.3 * 0.3
NSAMP = 16
B = 8
N = 2048
M = 2048
C = 64
L = 16
NWORKERS = 32
QPW = (B * N) // NWORKERS
WPB = N // QPW
NCHUNK = M // L


def _dense_body(s2_ref, x1_ref, p2_ref, p1_ref, wst_ref, wxt_ref, wdt_ref,
                bias_ref, g2_ref, q1_ref, p2s_ref, p1r_ref):
    s2 = s2_ref[0]
    x1 = x1_ref[0]
    p2 = p2_ref[0]
    p1 = p1_ref[0]
    wst = wst_ref[...]
    wxt = wxt_ref[...]
    wdt = wdt_ref[...]
    bias = bias_ref[...]

    dn = (((0,), (0,)), ((), ()))
    g2 = lax.dot_general(s2, wst, dn, preferred_element_type=jnp.float32)
    g2 = g2 + jnp.dot(p2, wdt, preferred_element_type=jnp.float32)
    q1 = lax.dot_general(x1, wxt, dn, preferred_element_type=jnp.float32)
    q1 = q1 - jnp.dot(p1, wdt, preferred_element_type=jnp.float32) + bias
    g2_ref[0] = g2
    q1_ref[0] = q1
    p2s_ref[0] = jnp.transpose(p2, (1, 0))
    p1r_ref[0] = jnp.pad(p1, ((0, 0), (0, L - 3)))


def _dense_stage(S2, X1, P2, P1, WsT, WxT, WdT, bias2d):
    grid = (B,)
    return pl.pallas_call(
        _dense_body,
        grid=grid,
        in_specs=[
            pl.BlockSpec((1, C, M), lambda b: (b, 0, 0)),
            pl.BlockSpec((1, C, N), lambda b: (b, 0, 0)),
            pl.BlockSpec((1, M, 3), lambda b: (b, 0, 0)),
            pl.BlockSpec((1, N, 3), lambda b: (b, 0, 0)),
            pl.BlockSpec((C, C), lambda b: (0, 0)),
            pl.BlockSpec((C, C), lambda b: (0, 0)),
            pl.BlockSpec((3, C), lambda b: (0, 0)),
            pl.BlockSpec((1, C), lambda b: (0, 0)),
        ],
        out_specs=[
            pl.BlockSpec((1, M, C), lambda b: (b, 0, 0)),
            pl.BlockSpec((1, N, C), lambda b: (b, 0, 0)),
            pl.BlockSpec((1, 3, M), lambda b: (b, 0, 0)),
            pl.BlockSpec((1, N, L), lambda b: (b, 0, 0)),
        ],
        out_shape=[
            jax.ShapeDtypeStruct((B, M, C), jnp.float32),
            jax.ShapeDtypeStruct((B, N, C), jnp.float32),
            jax.ShapeDtypeStruct((B, 3, M), jnp.float32),
            jax.ShapeDtypeStruct((B, N, L), jnp.float32),
        ],
    )(S2, X1, P2, P1, WsT, WxT, WdT, bias2d)


QCH = 8
NGCH = QPW // QCH
SPAN = 4
NSTEP = NCHUNK // SPAN
USTEPS = 8
IBUF2 = 96
TRASH = 95


def _sc_body(g2_hbm, p2s_hbm, p1w_hbm, q1_hbm, out_hbm,
             p2v, p1v, q1v, outv, ibuf, idxall, rows0, rows1,
             fbuf, sem0, sem1):
    wid = lax.axis_index("s") * 2 + lax.axis_index("c")
    b = wid // WPB
    g0 = wid * QPW

    pltpu.sync_copy(p2s_hbm.at[b], p2v)
    pltpu.sync_copy(p1w_hbm.at[wid], p1v)
    pltpu.sync_copy(q1_hbm.at[pl.ds(g0, QPW)], q1v)

    iota = jnp.arange(L, dtype=jnp.int32)
    bM = b * M

    BIG = jnp.int32(1 << 30)

    def scan_chunks(step, offv, qx, qy, qz):
        for k in range(SPAN):
            base = step * (SPAN * L) + k * L
            dx = p2v[0, pl.ds(base, L)] - qx
            dy = p2v[1, pl.ds(base, L)] - qy
            dz = p2v[2, pl.ds(base, L)] - qz
            w = dx * dx + dy * dy + dz * dz < RADIUS2
            cum = plsc.cumsum(w.astype(jnp.int32))
            pcnt = plsc.all_reduce_population_count(w)
            pos = jnp.where(w, jnp.minimum(offv + cum - 1, TRASH), TRASH)
            plsc.store_scatter(ibuf, [pos], base + iota)
            offv = offv + pcnt
        return offv

    def scan_query(q):
        pq = p1v[q, :]
        qx = jnp.full((L,), pq[0], jnp.float32)
        qy = jnp.full((L,), pq[1], jnp.float32)
        qz = jnp.full((L,), pq[2], jnp.float32)

        @plsc.parallel_loop(0, USTEPS, carry=jnp.zeros((L,), jnp.int32))
        def offv_final(step, offv):
            return scan_chunks(step, offv, qx, qy, qz)

        fbuf[0] = offv_final[0]

        def body1(cp, cr):
            @pl.when(fbuf[0] < NSAMP)
            def _():
                offv2 = jnp.full((L,), fbuf[0], jnp.int32)
                fbuf[0] = scan_chunks(cp, offv2, qx, qy, qz)[0]
            return cr

        @pl.when(fbuf[0] < NSAMP)
        def _tail():
            lax.fori_loop(USTEPS, NSTEP, body1, 0)

        found = fbuf[0]
        hits = ibuf[pl.ds(0, L)]
        first = jnp.where(found > 0, hits[0], 0)
        idx = jnp.where(iota < found, hits, first) + bM
        idxall[pl.ds(q * NSAMP, L)] = idx

    def scan_group(g):
        def one(qq, cr):
            scan_query(g * QCH + qq)
            return cr
        lax.fori_loop(0, QCH, one, 0)

    def issue(g, buf, sem):
        return pltpu.async_copy(
            g2_hbm.at[idxall.at[pl.ds(g * QCH * NSAMP, QCH * NSAMP)]],
            buf, sem)

    def wait(g, buf, sem):
        pltpu.make_async_copy(
            g2_hbm.at[idxall.at[pl.ds(g * QCH * NSAMP, QCH * NSAMP)]],
            buf, sem).wait()

    def process(g, buf):
        def one_query(qq, cr):
            r0 = qq * NSAMP
            qa = g * QCH + qq
            qav = jnp.full((L,), qa, jnp.int32)
            for cg in range(C // L):
                sl = pl.ds(cg * L, L)
                acc = buf[r0, sl]
                for j in range(1, NSAMP):
                    acc = jnp.maximum(acc, buf[r0 + j, sl])
                res = jnp.maximum(acc + q1v[qa, sl], 0.0)
                plsc.store_scatter(outv, [cg * L + iota, qav], res)
            return cr

        lax.fori_loop(0, QCH, one_query, 0)

    scan_group(0)
    issue(0, rows0, sem0)

    def pipe(s, carry):
        g = 2 * s + 1
        scan_group(g)
        issue(g, rows1, sem1)
        wait(g - 1, rows0, sem0)
        process(g - 1, rows0)
        scan_group(g + 1)
        issue(g + 1, rows0, sem0)
        wait(g, rows1, sem1)
        process(g, rows1)
        return carry

    lax.fori_loop(0, NGCH // 2 - 1, pipe, jnp.int32(0))
    scan_group(NGCH - 1)
    issue(NGCH - 1, rows1, sem1)
    wait(NGCH - 2, rows0, sem0)
    process(NGCH - 2, rows0)
    wait(NGCH - 1, rows1, sem1)
    process(NGCH - 1, rows1)
    n0 = (wid % WPB) * QPW
    pltpu.sync_copy(outv, out_hbm.at[b, :, pl.ds(n0, QPW)])


def _sc_stage(G2flat, P2s, P1w, Q1flat):
    mesh = plsc.VectorSubcoreMesh(core_axis_name="c", subcore_axis_name="s")
    kfn = pl.kernel(
        _sc_body,
        out_type=jax.ShapeDtypeStruct((B, C, N), jnp.float32),
        mesh=mesh,
        scratch_types=[
            pltpu.VMEM((3, M), jnp.float32),
            pltpu.VMEM((QPW, L), jnp.float32),
            pltpu.VMEM((QPW, C), jnp.float32),
            pltpu.VMEM((C, QPW), jnp.float32),
            pltpu.VMEM((IBUF2,), jnp.int32),
            pltpu.VMEM((QPW * NSAMP,), jnp.int32),
            pltpu.VMEM((QCH * NSAMP, C), jnp.float32),
            pltpu.VMEM((QCH * NSAMP, C), jnp.float32),
            pltpu.SMEM((1,), jnp.int32),
            pltpu.SemaphoreType.DMA,
            pltpu.SemaphoreType.DMA,
        ],
        compiler_params=pltpu.CompilerParams(needs_layout_passes=False,
                                             use_tc_tiling_on_sc=False),
    )
    return kfn(G2flat, P2s, P1w, Q1flat)


def kernel(P1, P2, X1, S2, W, b):
    WsT = jnp.transpose(W[:, :C])
    WxT = jnp.transpose(W[:, C:2 * C])
    WdT = jnp.transpose(W[:, 2 * C:])
    bias2d = b[None, :]

    G2, Q1, P2s, P1r = _dense_stage(S2, X1, P2, P1, WsT, WxT, WdT, bias2d)
    G2flat = G2.reshape(B * M, C)
    Q1flat = Q1.reshape(B * N, C)
    P1w = P1r.reshape(NWORKERS, QPW, L)

    return _sc_stage(G2flat, P2s, P1w, Q1flat)


# speedup vs baseline: 1.1984x; 1.1984x over previous
import functools

import jax
import jax.numpy as jnp
from jax import lax
from jax.experimental import pallas as pl
from jax.experimental.pallas import tpu as pltpu
from jax.experimental.pallas import tpu_sc as plsc

RADIUS2 = 0.3 * 0.3
NSAMP = 16
B = 8
N = 2048
M = 2048
C = 64
L = 16
NWORKERS = 32
QPW = (B * N) // NWORKERS
WPB = N // QPW
NCHUNK = M // L


def _dense_body(s2_ref, x1_ref, p2_ref, p1_ref, wst_ref, wxt_ref, wdt_ref,
                bias_ref, g2_ref, q1_ref, p2s_ref, p1r_ref):
    s2 = s2_ref[0]
    x1 = x1_ref[0]
    p2 = p2_ref[0]
    p1 = p1_ref[0]
    wst = wst_ref[...]
    wxt = wxt_ref[...]
    wdt = wdt_ref[...]
    bias = bias_ref[...]

    dn = (((0,), (0,)), ((), ()))
    g2 = lax.dot_general(s2, wst, dn, preferred_element_type=jnp.float32)
    g2 = g2 + jnp.dot(p2, wdt, preferred_element_type=jnp.float32)
    q1 = lax.dot_general(x1, wxt, dn, preferred_element_type=jnp.float32)
    q1 = q1 - jnp.dot(p1, wdt, preferred_element_type=jnp.float32) + bias
    g2_ref[0] = g2
    q1_ref[0] = q1
    p2s_ref[0] = jnp.transpose(p2, (1, 0))
    p1r_ref[0] = jnp.pad(p1, ((0, 0), (0, L - 3)))


def _dense_stage(S2, X1, P2, P1, WsT, WxT, WdT, bias2d):
    grid = (B,)
    return pl.pallas_call(
        _dense_body,
        grid=grid,
        in_specs=[
            pl.BlockSpec((1, C, M), lambda b: (b, 0, 0)),
            pl.BlockSpec((1, C, N), lambda b: (b, 0, 0)),
            pl.BlockSpec((1, M, 3), lambda b: (b, 0, 0)),
            pl.BlockSpec((1, N, 3), lambda b: (b, 0, 0)),
            pl.BlockSpec((C, C), lambda b: (0, 0)),
            pl.BlockSpec((C, C), lambda b: (0, 0)),
            pl.BlockSpec((3, C), lambda b: (0, 0)),
            pl.BlockSpec((1, C), lambda b: (0, 0)),
        ],
        out_specs=[
            pl.BlockSpec((1, M, C), lambda b: (b, 0, 0)),
            pl.BlockSpec((1, N, C), lambda b: (b, 0, 0)),
            pl.BlockSpec((1, 3, M), lambda b: (b, 0, 0)),
            pl.BlockSpec((1, N, L), lambda b: (b, 0, 0)),
        ],
        out_shape=[
            jax.ShapeDtypeStruct((B, M, C), jnp.float32),
            jax.ShapeDtypeStruct((B, N, C), jnp.float32),
            jax.ShapeDtypeStruct((B, 3, M), jnp.float32),
            jax.ShapeDtypeStruct((B, N, L), jnp.float32),
        ],
    )(S2, X1, P2, P1, WsT, WxT, WdT, bias2d)


QCH = 8
NGCH = QPW // QCH
SPAN = 4
NSTEP = NCHUNK // SPAN
PSTEPS = 4
USTEPS = 8
IBUF2 = 96
TRASH = 95


def _sc_body(g2_hbm, p2s_hbm, p1w_hbm, q1_hbm, out_hbm,
             p2v, p1v, q1v, outv, ibuf, idxall, rows0, rows1,
             fbuf, sem0, sem1):
    wid = lax.axis_index("s") * 2 + lax.axis_index("c")
    b = wid // WPB
    g0 = wid * QPW

    pltpu.sync_copy(p2s_hbm.at[b], p2v)
    pltpu.sync_copy(p1w_hbm.at[wid], p1v)
    pltpu.sync_copy(q1_hbm.at[pl.ds(g0, QPW)], q1v)

    iota = jnp.arange(L, dtype=jnp.int32)
    bM = b * M

    BIG = jnp.int32(1 << 30)

    def scan_chunks(step, offv, qx, qy, qz):
        for k in range(SPAN):
            base = step * (SPAN * L) + k * L
            dx = p2v[0, pl.ds(base, L)] - qx
            dy = p2v[1, pl.ds(base, L)] - qy
            dz = p2v[2, pl.ds(base, L)] - qz
            w = dx * dx + dy * dy + dz * dz < RADIUS2
            cum = plsc.cumsum(w.astype(jnp.int32))
            pcnt = plsc.all_reduce_population_count(w)
            pos = jnp.where(w, jnp.minimum(offv + cum - 1, TRASH), TRASH)
            plsc.store_scatter(ibuf, [pos], base + iota)
            offv = offv + pcnt
        return offv

    def scan_query(q):
        pq = p1v[q, :]
        qx = jnp.full((L,), pq[0], jnp.float32)
        qy = jnp.full((L,), pq[1], jnp.float32)
        qz = jnp.full((L,), pq[2], jnp.float32)

        @plsc.parallel_loop(0, PSTEPS, carry=jnp.zeros((L,), jnp.int32))
        def offv_final(step, offv):
            return scan_chunks(step, offv, qx, qy, qz)

        fbuf[0] = offv_final[0]

        def body1(cp, cr):
            @pl.when(fbuf[0] < NSAMP)
            def _():
                offv2 = jnp.full((L,), fbuf[0], jnp.int32)
                fbuf[0] = scan_chunks(cp, offv2, qx, qy, qz)[0]
            return cr

        lax.fori_loop(PSTEPS, USTEPS, body1, 0)

        @pl.when(fbuf[0] < NSAMP)
        def _tail():
            lax.fori_loop(USTEPS, NSTEP, body1, 0)

        found = fbuf[0]
        hits = ibuf[pl.ds(0, L)]
        first = jnp.where(found > 0, hits[0], 0)
        idx = jnp.where(iota < found, hits, first) + bM
        idxall[pl.ds(q * NSAMP, L)] = idx

    def scan_group(g):
        def one(qq, cr):
            scan_query(g * QCH + qq)
            return cr
        lax.fori_loop(0, QCH, one, 0)

    def issue(g, buf, sem):
        return pltpu.async_copy(
            g2_hbm.at[idxall.at[pl.ds(g * QCH * NSAMP, QCH * NSAMP)]],
            buf, sem)

    def wait(g, buf, sem):
        pltpu.make_async_copy(
            g2_hbm.at[idxall.at[pl.ds(g * QCH * NSAMP, QCH * NSAMP)]],
            buf, sem).wait()

    def process(g, buf):
        def one_query(qq, cr):
            r0 = qq * NSAMP
            qa = g * QCH + qq
            qav = jnp.full((L,), qa, jnp.int32)
            for cg in range(C // L):
                sl = pl.ds(cg * L, L)
                acc = buf[r0, sl]
                for j in range(1, NSAMP):
                    acc = jnp.maximum(acc, buf[r0 + j, sl])
                res = jnp.maximum(acc + q1v[qa, sl], 0.0)
                plsc.store_scatter(outv, [cg * L + iota, qav], res)
            return cr

        lax.fori_loop(0, QCH, one_query, 0)

    scan_group(0)
    issue(0, rows0, sem0)

    def pipe(s, carry):
        g = 2 * s + 1
        scan_group(g)
        issue(g, rows1, sem1)
        wait(g - 1, rows0, sem0)
        process(g - 1, rows0)
        scan_group(g + 1)
        issue(g + 1, rows0, sem0)
        wait(g, rows1, sem1)
        process(g, rows1)
        return carry

    lax.fori_loop(0, NGCH // 2 - 1, pipe, jnp.int32(0))
    scan_group(NGCH - 1)
    issue(NGCH - 1, rows1, sem1)
    wait(NGCH - 2, rows0, sem0)
    process(NGCH - 2, rows0)
    wait(NGCH - 1, rows1, sem1)
    process(NGCH - 1, rows1)
    n0 = (wid % WPB) * QPW
    pltpu.sync_copy(outv, out_hbm.at[b, :, pl.ds(n0, QPW)])


def _sc_stage(G2flat, P2s, P1w, Q1flat):
    mesh = plsc.VectorSubcoreMesh(core_axis_name="c", subcore_axis_name="s")
    kfn = pl.kernel(
        _sc_body,
        out_type=jax.ShapeDtypeStruct((B, C, N), jnp.float32),
        mesh=mesh,
        scratch_types=[
            pltpu.VMEM((3, M), jnp.float32),
            pltpu.VMEM((QPW, L), jnp.float32),
            pltpu.VMEM((QPW, C), jnp.float32),
            pltpu.VMEM((C, QPW), jnp.float32),
            pltpu.VMEM((IBUF2,), jnp.int32),
            pltpu.VMEM((QPW * NSAMP,), jnp.int32),
            pltpu.VMEM((QCH * NSAMP, C), jnp.float32),
            pltpu.VMEM((QCH * NSAMP, C), jnp.float32),
            pltpu.SMEM((1,), jnp.int32),
            pltpu.SemaphoreType.DMA,
            pltpu.SemaphoreType.DMA,
        ],
        compiler_params=pltpu.CompilerParams(needs_layout_passes=False,
                                             use_tc_tiling_on_sc=False),
    )
    return kfn(G2flat, P2s, P1w, Q1flat)


def kernel(P1, P2, X1, S2, W, b):
    WsT = jnp.transpose(W[:, :C])
    WxT = jnp.transpose(W[:, C:2 * C])
    WdT = jnp.transpose(W[:, 2 * C:])
    bias2d = b[None, :]

    G2, Q1, P2s, P1r = _dense_stage(S2, X1, P2, P1, WsT, WxT, WdT, bias2d)
    G2flat = G2.reshape(B * M, C)
    Q1flat = Q1.reshape(B * N, C)
    P1w = P1r.reshape(NWORKERS, QPW, L)

    return _sc_stage(G2flat, P2s, P1w, Q1flat)
